# baseline (speedup 1.0000x reference)
import jax
import jax.numpy as jnp
from jax.experimental import pallas as pl


_BG = 1000


def _body(x_ref, u_ref, o_ref):
    x0 = x_ref[:, 0, :]
    x1 = x_ref[:, 1, :]
    u0 = jnp.clip(u_ref[:, 0, :], 1e-6, 1.0 - 1e-6)
    u1 = jnp.clip(u_ref[:, 1, :], 1e-6, 1.0 - 1e-6)
    s0 = x0 - jnp.log(-jnp.log(u0))
    s1 = x1 - jnp.log(-jnp.log(u1))
    o_ref[...] = (s1 > s0).astype(jnp.int32)


def _plane_view(a, g):
    return a.reshape(g, 128, 2).transpose(0, 2, 1)


def kernel(edge_logp, noise_u):
    n = edge_logp.shape[0]
    g = n // 128
    x = _plane_view(edge_logp, g)
    u = _plane_view(noise_u, g)
    grid = g // _BG
    spec = pl.BlockSpec((_BG, 2, 128), lambda i: (i, 0, 0))
    out = pl.pallas_call(
        _body,
        grid=(grid,),
        in_specs=[spec, spec],
        out_specs=pl.BlockSpec((_BG, 128), lambda i: (i, 0)),
        out_shape=jax.ShapeDtypeStruct((g, 128), jnp.int32),
    )(x, u)
    return out.reshape(n)


# speedup vs baseline: 2.1197x; 2.1197x over previous
import jax
import jax.numpy as jnp
from jax.experimental import pallas as pl
from jax.experimental.pallas import tpu as pltpu


_BR = 1000


def _body(x_ref, u_ref, o_ref):
    x = x_ref[...]
    u = jnp.clip(u_ref[...], 1e-6, 1.0 - 1e-6)
    s = x - jnp.log(-jnp.log(u))
    t = pltpu.roll(s, shift=s.shape[0] - 1, axis=0)
    c = (t > s).astype(jnp.int32)
    c3 = c.reshape(c.shape[0] // 2, 2, 128)
    o_ref[...] = c3[:, 0, :]


def _flat_view(a, n):
    g = n // 128
    return a.reshape(g, 128, 2).transpose(0, 2, 1).reshape(2 * g, 128)


def kernel(edge_logp, noise_u):
    n = edge_logp.shape[0]
    g = n // 128
    x = _flat_view(edge_logp, n)
    u = _flat_view(noise_u, n)
    grid = g // _BR
    out = pl.pallas_call(
        _body,
        grid=(grid,),
        in_specs=[
            pl.BlockSpec((2 * _BR, 128), lambda i: (i, 0)),
            pl.BlockSpec((2 * _BR, 128), lambda i: (i, 0)),
        ],
        out_specs=pl.BlockSpec((_BR, 128), lambda i: (i, 0)),
        out_shape=jax.ShapeDtypeStruct((g, 128), jnp.int32),
    )(x, u)
    return out.reshape(n)


# speedup vs baseline: 2.5876x; 1.2208x over previous
import jax
import jax.numpy as jnp
from jax.experimental import pallas as pl
from jax.experimental.pallas import tpu as pltpu


_BR = 2000


def _body(x_ref, u_ref, o_ref):
    x = x_ref[...]
    u = jnp.clip(u_ref[...], 1e-6, 1.0 - 1e-6)
    s = x - jnp.log(-jnp.log(u))
    t = pltpu.roll(s, shift=s.shape[0] - 1, axis=0)
    c = (t > s).astype(jnp.int32)
    c3 = c.reshape(c.shape[0] // 2, 2, 128)
    o_ref[...] = c3[:, 0, :]


def _flat_view(a, n):
    g = n // 128
    return a.reshape(g, 128, 2).transpose(0, 2, 1).reshape(2 * g, 128)


def kernel(edge_logp, noise_u):
    n = edge_logp.shape[0]
    g = n // 128
    x = _flat_view(edge_logp, n)
    u = _flat_view(noise_u, n)
    grid = g // _BR
    out = pl.pallas_call(
        _body,
        grid=(grid,),
        in_specs=[
            pl.BlockSpec((2 * _BR, 128), lambda i: (i, 0)),
            pl.BlockSpec((2 * _BR, 128), lambda i: (i, 0)),
        ],
        out_specs=pl.BlockSpec((_BR, 128), lambda i: (i, 0)),
        out_shape=jax.ShapeDtypeStruct((g, 128), jnp.int32),
    )(x, u)
    return out.reshape(n)
